# Initial kernel scaffold; baseline (speedup 1.0000x reference)
import jax
import jax.numpy as jnp
from jax.experimental import pallas as pl


def kernel(node_ids, adj, mask, emb, W0, b0, W1, b1, W2, b2, Wf1, bf1, Wf2, bf2):
    raise NotImplementedError("write your pallas kernel here")


# speedup vs baseline: 1.7513x; 1.7513x over previous
import functools

import jax
import jax.numpy as jnp
from jax import lax
from jax.experimental import pallas as pl
from jax.experimental.pallas import tpu as pltpu
from jax.experimental.pallas import tpu_sc as plsc

_B, _N, _VOCAB = 32, 512, 10000
_IN = 512
_H0, _H1, _H2 = 128, 64, 32
_OUT = 128
_GPS = 8
_CHUNK = 128


def _proj_body(emb_ref, w_ref, out_ref):
    out_ref[...] = lax.dot_general(
        emb_ref[...], w_ref[...], (((1,), (1,)), ((), ())),
        preferred_element_type=jnp.float32)


def _project_table(emb, w0):
    tile = 2000
    grid = _VOCAB // tile
    return pl.pallas_call(
        _proj_body,
        grid=(grid,),
        in_specs=[
            pl.BlockSpec((tile, _IN), lambda i: (i, 0)),
            pl.BlockSpec((_H0, _IN), lambda i: (0, 0)),
        ],
        out_specs=pl.BlockSpec((tile, _H0), lambda i: (i, 0)),
        out_shape=jax.ShapeDtypeStruct((_VOCAB, _H0), jnp.float32),
    )(emb, w0)


def _gather_rows(table, idx2d):
    info = plsc.get_sparse_core_info()
    nw = info.num_cores * info.num_subcores
    tot = idx2d.shape[0] * idx2d.shape[1]
    b_per_w = tot // nw
    chunks = b_per_w // _CHUNK
    mesh = plsc.VectorSubcoreMesh(core_axis_name="c", subcore_axis_name="s")

    @functools.partial(
        pl.kernel,
        out_type=jax.ShapeDtypeStruct((tot, _H0), jnp.float32),
        mesh=mesh,
        scratch_types=[
            pltpu.VMEM((chunks, _CHUNK), jnp.int32),
            pltpu.VMEM((b_per_w, _H0), jnp.float32),
            pltpu.SemaphoreType.DMA,
        ],
    )
    def gather_kernel(table_hbm, idx_hbm, out_hbm, idx_v, rows_v, sem):
        wid = lax.axis_index("s") * info.num_cores + lax.axis_index("c")
        row0 = wid * chunks
        pltpu.sync_copy(idx_hbm.at[pl.ds(row0, chunks)], idx_v)
        copies = [
            pltpu.async_copy(
                table_hbm.at[idx_v.at[j]],
                rows_v.at[pl.ds(j * _CHUNK, _CHUNK)],
                sem,
            )
            for j in range(chunks)
        ]
        for c in copies:
            c.wait()
        pltpu.sync_copy(rows_v, out_hbm.at[pl.ds(wid * b_per_w, b_per_w)])

    return gather_kernel(table, idx2d)


def _gcn_body(adj_ref, x0_ref, mask_ref, b0_ref, w1_ref, b1_ref, w2_ref,
              b2_ref, wf1_ref, bf1_ref, wf2_ref, bf2_ref, out_ref):
    def mm(x, w):
        return lax.dot_general(x, w, (((1,), (1,)), ((), ())),
                               preferred_element_type=jnp.float32)

    for g in range(_GPS):
        a = adj_ref[g]
        asym = jnp.logical_or(a != 0.0, a.T != 0.0).astype(jnp.float32)
        m = mask_ref[g]
        h = (jnp.dot(asym, x0_ref[g], preferred_element_type=jnp.float32)
             + b0_ref[...]) * m
        h = (jnp.dot(asym, mm(h, w1_ref[...]),
                     preferred_element_type=jnp.float32) + b1_ref[...]) * m
        h = (jnp.dot(asym, mm(h, w2_ref[...]),
                     preferred_element_type=jnp.float32) + b2_ref[...]) * m
        p = jnp.max(h, axis=0, keepdims=True)
        f = mm(p, wf1_ref[...]) + bf1_ref[...]
        o = mm(f, wf2_ref[...]) + bf2_ref[...]
        out_ref[pl.ds(g, 1), :] = jnp.maximum(o, 0.0)


def _gcn(adj, x0, mask3, b0, w1, b1, w2, b2, wf1, bf1, wf2, bf2):
    grid = _B // _GPS
    full = lambda shape: pl.BlockSpec(shape, lambda i: tuple(0 for _ in shape))
    return pl.pallas_call(
        _gcn_body,
        grid=(grid,),
        in_specs=[
            pl.BlockSpec((_GPS, _N, _N), lambda i: (i, 0, 0)),
            pl.BlockSpec((_GPS, _N, _H0), lambda i: (i, 0, 0)),
            pl.BlockSpec((_GPS, _N, 1), lambda i: (i, 0, 0)),
            full((1, _H0)),
            full((_H1, _H0)),
            full((1, _H1)),
            full((_H2, _H1)),
            full((1, _H2)),
            full((_H2, _H2)),
            full((1, _H2)),
            full((_OUT, _H2)),
            full((1, _OUT)),
        ],
        out_specs=pl.BlockSpec((_GPS, _OUT), lambda i: (i, 0)),
        out_shape=jax.ShapeDtypeStruct((_B, _OUT), jnp.float32),
        compiler_params=pltpu.CompilerParams(
            dimension_semantics=("arbitrary",)),
    )(adj, x0, mask3, b0, w1, b1, w2, b2, wf1, bf1, wf2, bf2)


def kernel(node_ids, adj, mask, emb, W0, b0, W1, b1, W2, b2, Wf1, bf1, Wf2,
           bf2):
    embw = _project_table(emb, W0)
    idx2d = node_ids.astype(jnp.int32).reshape(-1, _CHUNK)
    x0 = _gather_rows(embw, idx2d).reshape(_B, _N, _H0)
    return _gcn(adj, x0, mask[:, :, None], b0.reshape(1, -1), W1,
                b1.reshape(1, -1), W2, b2.reshape(1, -1), Wf1,
                bf1.reshape(1, -1), Wf2, bf2.reshape(1, -1))
